# Initial kernel scaffold; baseline (speedup 1.0000x reference)
import jax
import jax.numpy as jnp
from jax.experimental import pallas as pl


def kernel(feature, table):
    raise NotImplementedError("write your pallas kernel here")


# speedup vs baseline: 1.8435x; 1.8435x over previous
import functools

import jax
import jax.numpy as jnp
from jax import lax
from jax.experimental import pallas as pl
from jax.experimental.pallas import tpu as pltpu
from jax.experimental.pallas import tpu_sc as plsc


@functools.cache
def _make_gather(V, D, B):
    info = plsc.get_sparse_core_info()
    NC, NS = info.num_cores, info.num_subcores
    NW = NC * NS
    assert B % NW == 0
    b_per_w = B // NW
    chunk = 1024
    assert b_per_w % chunk == 0
    n_chunks = b_per_w // chunk
    mesh = plsc.VectorSubcoreMesh(core_axis_name="c", subcore_axis_name="s")

    @functools.partial(
        pl.kernel,
        mesh=mesh,
        out_type=jax.ShapeDtypeStruct((B, D), jnp.float32),
        scratch_types=[
            pltpu.VMEM((chunk,), jnp.int32),
            pltpu.VMEM((chunk, D), jnp.float32),
            pltpu.SemaphoreType.DMA,
        ],
        compiler_params=pltpu.CompilerParams(use_tc_tiling_on_sc=False),
    )
    def gather_kernel(idx_hbm, table_hbm, out_hbm, idx_v, rows_v, sem):
        wid = lax.axis_index("s") * NC + lax.axis_index("c")
        base = wid * b_per_w

        def body(i, carry):
            off = base + i * chunk
            pltpu.sync_copy(idx_hbm.at[pl.ds(off, chunk)], idx_v)
            pltpu.async_copy(table_hbm.at[idx_v], rows_v, sem).wait()
            pltpu.sync_copy(rows_v, out_hbm.at[pl.ds(off, chunk)])
            return carry

        lax.fori_loop(0, n_chunks, body, 0)

    return gather_kernel


def kernel(feature, table):
    batch, hist = feature.shape
    dim = table.shape[1]
    idx = feature.reshape(-1).astype(jnp.int32)
    out = _make_gather(table.shape[0], dim, idx.shape[0])(idx, table)
    return out.reshape(batch, hist, dim)


# speedup vs baseline: 1.8644x; 1.0113x over previous
import functools

import jax
import jax.numpy as jnp
from jax import lax
from jax.experimental import pallas as pl
from jax.experimental.pallas import tpu as pltpu
from jax.experimental.pallas import tpu_sc as plsc

_CHUNK = 512
_NBUF = 2


@functools.cache
def _make_gather(V, D, B):
    info = plsc.get_sparse_core_info()
    NC, NS = info.num_cores, info.num_subcores
    NW = NC * NS
    assert B % NW == 0
    b_per_w = B // NW
    chunk, nbuf = _CHUNK, _NBUF
    assert b_per_w % (chunk * nbuf) == 0
    n_groups = b_per_w // (chunk * nbuf)
    mesh = plsc.VectorSubcoreMesh(core_axis_name="c", subcore_axis_name="s")

    @functools.partial(
        pl.kernel,
        mesh=mesh,
        out_type=jax.ShapeDtypeStruct((B, D), jnp.float32),
        scratch_types=[
            pltpu.VMEM((b_per_w,), jnp.int32),
            [pltpu.VMEM((chunk, D), jnp.float32) for _ in range(nbuf)],
            [pltpu.SemaphoreType.DMA for _ in range(nbuf)],
            [pltpu.SemaphoreType.DMA for _ in range(nbuf)],
        ],
        compiler_params=pltpu.CompilerParams(use_tc_tiling_on_sc=False),
    )
    def gather_kernel(idx_hbm, table_hbm, out_hbm, idx_v, rbs, gsems, ssems):
        wid = lax.axis_index("s") * NC + lax.axis_index("c")
        base = wid * b_per_w
        pltpu.sync_copy(idx_hbm.at[pl.ds(base, b_per_w)], idx_v)

        def start_gather(coff, b):
            return pltpu.async_copy(
                table_hbm.at[idx_v.at[pl.ds(coff, chunk)]], rbs[b], gsems[b]
            )

        def start_store(coff, b):
            return pltpu.async_copy(
                rbs[b], out_hbm.at[pl.ds(base + coff, chunk)], ssems[b]
            )

        def wait_gather(b):
            pltpu.make_async_copy(
                table_hbm.at[idx_v.at[pl.ds(0, chunk)]], rbs[b], gsems[b]
            ).wait()

        def wait_store(b):
            pltpu.make_async_copy(
                rbs[b], out_hbm.at[pl.ds(base, chunk)], ssems[b]
            ).wait()

        for b in range(nbuf):
            start_gather(b * chunk, b)
        for b in range(nbuf):
            wait_gather(b)
            start_store(b * chunk, b)

        def group(g, carry):
            goff = g * nbuf * chunk
            for b in range(nbuf):
                wait_store(b)
                start_gather(goff + b * chunk, b)
            for b in range(nbuf):
                wait_gather(b)
                start_store(goff + b * chunk, b)
            return carry

        lax.fori_loop(1, n_groups, group, 0)
        for b in range(nbuf):
            wait_store(b)

    return gather_kernel


def kernel(feature, table):
    batch, hist = feature.shape
    dim = table.shape[1]
    idx = feature.reshape(-1).astype(jnp.int32)
    out = _make_gather(table.shape[0], dim, idx.shape[0])(idx, table)
    return out.reshape(batch, hist, dim)
